# Initial kernel scaffold; baseline (speedup 1.0000x reference)
import jax
import jax.numpy as jnp
from jax.experimental import pallas as pl


def kernel(x, edge_index, W1, b1, W2, b2):
    raise NotImplementedError("write your pallas kernel here")


# speedup vs baseline: 11.7041x; 11.7041x over previous
import functools

import jax
import jax.numpy as jnp
from jax import lax
from jax.experimental import pallas as pl
from jax.experimental.pallas import tpu as pltpu
from jax.experimental.pallas import tpu_sc as plsc

N = 10000
E = 320000
NF = 128
NH = 128
NT = 64

NC, NS, L = 2, 16, 16
NW = NC * NS
EPW = E // NW
EB = 80
NBLK = EPW // EB
RPT = 624
ZR = 48
TAIL0 = NS * RPT
TAILR = N - TAIL0

_MESH = dict(core_axis_name="c", subcore_axis_name="s")


@functools.partial(
    pl.kernel,
    out_type=jax.ShapeDtypeStruct((NW, N), jnp.float32),
    mesh=plsc.VectorSubcoreMesh(**_MESH),
    compiler_params=pltpu.CompilerParams(needs_layout_passes=False),
    scratch_types=[
        pltpu.VMEM((EB,), jnp.int32),
        pltpu.VMEM((N,), jnp.float32),
    ],
)
def _deg_kernel(col_hbm, hist_out, idx_v, hist_v):
    cid = lax.axis_index("c")
    sid = lax.axis_index("s")
    wid = cid * NS + sid
    z = jnp.zeros((L,), jnp.float32)

    def zero_blk(i, _):
        hist_v[pl.ds(i * L, L)] = z
        return 0

    lax.fori_loop(0, N // L, zero_blk, 0)

    ones = jnp.full((L,), 1.0, jnp.float32)
    ebase = wid * EPW

    def blk(j, _):
        pltpu.sync_copy(col_hbm.at[pl.ds(ebase + j * EB, EB)], idx_v)
        for g in range(EB // L):
            idx = idx_v[pl.ds(g * L, L)]
            plsc.addupdate_scatter(hist_v, [idx], ones)
        return 0

    lax.fori_loop(0, NBLK, blk, 0)
    pltpu.sync_copy(hist_v, hist_out.at[wid])


def _make_agg(D):
    @functools.partial(
        pl.kernel,
        out_type=jax.ShapeDtypeStruct((NC, N, D), jnp.float32),
        mesh=plsc.VectorSubcoreMesh(**_MESH),
        compiler_params=pltpu.CompilerParams(use_tc_tiling_on_sc=False),
        scratch_types=[
            pltpu.VMEM_SHARED((N, D), jnp.float32),
            pltpu.VMEM((EB,), jnp.int32),
            pltpu.VMEM((EB,), jnp.int32),
            pltpu.VMEM((EB, D), jnp.float32),
            pltpu.VMEM((ZR, D), jnp.float32),
            pltpu.SemaphoreType.DMA,
        ],
    )
    def agg(g_hbm, row_hbm, col_hbm, out_hbm, acc_sh, ridx_v, cidx_v, rows_v,
            zrow_v, sem):
        cid = lax.axis_index("c")
        sid = lax.axis_index("s")
        wid = cid * NS + sid
        z = jnp.zeros((L,), jnp.float32)

        def zrow_blk(r, _):
            for c0 in range(D // L):
                zrow_v[r, pl.ds(c0 * L, L)] = z
            return 0

        lax.fori_loop(0, ZR, zrow_blk, 0)
        row0 = sid * RPT

        def zero_blk(i, _):
            pltpu.sync_copy(zrow_v, acc_sh.at[pl.ds(row0 + i * ZR, ZR)])
            return 0

        lax.fori_loop(0, RPT // ZR, zero_blk, 0)

        @pl.when(sid == 0)
        def _():
            pltpu.sync_copy(zrow_v.at[pl.ds(0, TAILR)],
                            acc_sh.at[pl.ds(TAIL0, TAILR)])

        plsc.subcore_barrier()

        ebase = wid * EPW

        def blk(j, _):
            eo = ebase + j * EB
            pltpu.sync_copy(row_hbm.at[pl.ds(eo, EB)], ridx_v)
            pltpu.sync_copy(col_hbm.at[pl.ds(eo, EB)], cidx_v)
            pltpu.async_copy(g_hbm.at[ridx_v], rows_v, sem).wait()
            pltpu.sync_copy(rows_v, acc_sh.at[cidx_v], add=True)
            return 0

        lax.fori_loop(0, NBLK, blk, 0)
        plsc.subcore_barrier()

        def copy_blk(i, _):
            r = row0 + i * ZR
            pltpu.sync_copy(acc_sh.at[pl.ds(r, ZR)],
                            out_hbm.at[cid, pl.ds(r, ZR)])
            return 0

        lax.fori_loop(0, RPT // ZR, copy_blk, 0)

        @pl.when(sid == 0)
        def _():
            pltpu.sync_copy(acc_sh.at[pl.ds(TAIL0, TAILR)],
                            out_hbm.at[cid, pl.ds(TAIL0, TAILR)])

    return agg


_agg128 = _make_agg(NH)
_agg64 = _make_agg(NT)


def _dense1_body(x_ref, w_ref, hist_ref, g_ref, dinv_ref):
    deg = jnp.sum(hist_ref[...], axis=0)
    dinv = jnp.where(deg > 0, 1.0 / jnp.sqrt(deg), 0.0)[:, None]
    h = jnp.dot(x_ref[...], w_ref[...], preferred_element_type=jnp.float32)
    g_ref[...] = h * dinv
    dinv_ref[...] = dinv


_dense1 = pl.pallas_call(
    _dense1_body,
    out_shape=[
        jax.ShapeDtypeStruct((N, NH), jnp.float32),
        jax.ShapeDtypeStruct((N, 1), jnp.float32),
    ],
)


def _dense2_body(p_ref, dinv_ref, b1_ref, w2_ref, g2_ref):
    dinv = dinv_ref[...]
    s = (p_ref[0] + p_ref[1]) * dinv + b1_ref[...][None, :]
    h = jnp.maximum(s, 0.0)
    g2_ref[...] = jnp.dot(h, w2_ref[...],
                          preferred_element_type=jnp.float32) * dinv


_dense2 = pl.pallas_call(
    _dense2_body,
    out_shape=jax.ShapeDtypeStruct((N, NT), jnp.float32),
)


def _final_body(p_ref, dinv_ref, b2_ref, o_ref):
    o_ref[...] = (p_ref[0] + p_ref[1]) * dinv_ref[...] + b2_ref[...][None, :]


_final = pl.pallas_call(
    _final_body,
    out_shape=jax.ShapeDtypeStruct((N, NT), jnp.float32),
)


def kernel(x, edge_index, W1, b1, W2, b2):
    row = edge_index[0]
    col = edge_index[1]
    hist = _deg_kernel(col)
    g1, dinv = _dense1(x, W1, hist)
    p1 = _agg128(g1, row, col)
    g2 = _dense2(p1, dinv, b1, W2)
    p2 = _agg64(g2, row, col)
    return _final(p2, dinv, b2)


# speedup vs baseline: 28.3339x; 2.4209x over previous
import functools

import jax
import jax.numpy as jnp
from jax import lax
from jax.experimental import pallas as pl
from jax.experimental.pallas import tpu as pltpu
from jax.experimental.pallas import tpu_sc as plsc

N = 10000
E = 320000
NF = 128
NH = 128
NT = 64

NC, NS, L = 2, 16, 16
NW = NC * NS
EPW = E // NW
EB = 80
NBLK = EPW // EB
EPS = E // NS
ABLK = EPS // EB
NB = 5
RPT = 624
ZR = 48
TAIL0 = NS * RPT
TAILR = N - TAIL0

_MESH = dict(core_axis_name="c", subcore_axis_name="s")


@functools.partial(
    pl.kernel,
    out_type=jax.ShapeDtypeStruct((NW, N), jnp.float32),
    mesh=plsc.VectorSubcoreMesh(**_MESH),
    compiler_params=pltpu.CompilerParams(needs_layout_passes=False),
    scratch_types=[
        pltpu.VMEM((NBLK, EB), jnp.int32),
        pltpu.VMEM((N,), jnp.float32),
        pltpu.SemaphoreType.DMA,
    ],
)
def _deg_kernel(col_hbm, hist_out, cidx_v, hist_v, sem):
    cid = lax.axis_index("c")
    sid = lax.axis_index("s")
    wid = cid * NS + sid
    idx_cp = pltpu.async_copy(col_hbm.at[wid], cidx_v, sem)
    z = jnp.zeros((L,), jnp.float32)

    def zero_blk(i, _):
        hist_v[pl.ds(i * L, L)] = z
        return 0

    lax.fori_loop(0, N // L, zero_blk, 0)
    idx_cp.wait()

    ones = jnp.full((L,), 1.0, jnp.float32)

    def blk(j, _):
        for g in range(EB // L):
            idx = cidx_v[j, pl.ds(g * L, L)]
            plsc.addupdate_scatter(hist_v, [idx], ones)
        return 0

    lax.fori_loop(0, NBLK, blk, 0)
    pltpu.sync_copy(hist_v, hist_out.at[wid])


def _make_agg(D):
    @functools.partial(
        pl.kernel,
        out_type=jax.ShapeDtypeStruct((NC, N, D), jnp.float32),
        mesh=plsc.VectorSubcoreMesh(**_MESH),
        compiler_params=pltpu.CompilerParams(use_tc_tiling_on_sc=False),
        scratch_types=[
            pltpu.VMEM_SHARED((N, D), jnp.float32),
            pltpu.VMEM((ABLK, EB), jnp.int32),
            pltpu.VMEM((ABLK, EB), jnp.int32),
            pltpu.VMEM((NB, EB, D), jnp.float32),
            pltpu.VMEM((ZR, D), jnp.float32),
            [pltpu.SemaphoreType.DMA] * NB,
            [pltpu.SemaphoreType.DMA] * NB,
            pltpu.SemaphoreType.DMA,
            pltpu.SemaphoreType.DMA,
        ],
    )
    def agg(ga_hbm, gb_hbm, row_hbm, col_hbm, out_hbm, acc_sh, ridx_v,
            cidx_v, rows_v, zrow_v, gsems, ssems, isem, jsem):
        cid = lax.axis_index("c")
        sid = lax.axis_index("s")
        rcp = pltpu.async_copy(row_hbm.at[sid], ridx_v, isem)
        ccp = pltpu.async_copy(col_hbm.at[sid], cidx_v, jsem)
        z = jnp.zeros((L,), jnp.float32)

        def zrow_blk(r, _):
            for c0 in range(D // L):
                zrow_v[r, pl.ds(c0 * L, L)] = z
            return 0

        lax.fori_loop(0, ZR, zrow_blk, 0)
        row0 = sid * RPT

        def zero_blk(i, _):
            pltpu.sync_copy(zrow_v, acc_sh.at[pl.ds(row0 + i * ZR, ZR)])
            return 0

        lax.fori_loop(0, RPT // ZR, zero_blk, 0)

        @pl.when(sid == 0)
        def _():
            pltpu.sync_copy(zrow_v.at[pl.ds(0, TAILR)],
                            acc_sh.at[pl.ds(TAIL0, TAILR)])

        rcp.wait()
        ccp.wait()

        def gather_start(j, b):
            @pl.when(cid == 0)
            def _():
                pltpu.async_copy(ga_hbm.at[ridx_v.at[j]], rows_v.at[b],
                                 gsems[b])

            @pl.when(cid == 1)
            def _():
                pltpu.async_copy(gb_hbm.at[ridx_v.at[j]], rows_v.at[b],
                                 gsems[b])

        for b in range(NB):
            gather_start(b, b)
        plsc.subcore_barrier()

        def blk(g, _):
            scat = []
            for b in range(NB):
                j = g * NB + b
                pltpu.make_async_copy(ga_hbm.at[ridx_v.at[j]], rows_v.at[b],
                                      gsems[b]).wait()
                scat.append(pltpu.async_copy(rows_v.at[b],
                                             acc_sh.at[cidx_v.at[j]],
                                             ssems[b], add=True))
            for b in range(NB):
                scat[b].wait()

                @pl.when(g < ABLK // NB - 1)
                def _():
                    gather_start(g * NB + b + NB, b)

            return 0

        lax.fori_loop(0, ABLK // NB, blk, 0)
        plsc.subcore_barrier()

        def copy_blk(i, _):
            r = row0 + i * ZR
            pltpu.sync_copy(acc_sh.at[pl.ds(r, ZR)],
                            out_hbm.at[cid, pl.ds(r, ZR)])
            return 0

        lax.fori_loop(0, RPT // ZR, copy_blk, 0)

        @pl.when(sid == 0)
        def _():
            pltpu.sync_copy(acc_sh.at[pl.ds(TAIL0, TAILR)],
                            out_hbm.at[cid, pl.ds(TAIL0, TAILR)])

    return agg


_agg1 = _make_agg(NH // 2)
_agg2 = _make_agg(NT // 2)


def _dense1_body(x_ref, w_ref, hist_ref, ga_ref, gb_ref, dinv_ref):
    deg = jnp.sum(hist_ref[...], axis=0)
    dinv = jnp.where(deg > 0, 1.0 / jnp.sqrt(deg), 0.0)[:, None]
    g = jnp.dot(x_ref[...], w_ref[...],
                preferred_element_type=jnp.float32) * dinv
    ga_ref[...] = g[:, : NH // 2]
    gb_ref[...] = g[:, NH // 2:]
    dinv_ref[...] = dinv


_dense1 = pl.pallas_call(
    _dense1_body,
    out_shape=[
        jax.ShapeDtypeStruct((N, NH // 2), jnp.float32),
        jax.ShapeDtypeStruct((N, NH // 2), jnp.float32),
        jax.ShapeDtypeStruct((N, 1), jnp.float32),
    ],
)


def _dense2_body(p_ref, dinv_ref, b1_ref, w2_ref, g2a_ref, g2b_ref):
    dinv = dinv_ref[...]
    p = jnp.concatenate([p_ref[0], p_ref[1]], axis=1)
    h = jnp.maximum(p * dinv + b1_ref[...][None, :], 0.0)
    g2 = jnp.dot(h, w2_ref[...], preferred_element_type=jnp.float32) * dinv
    g2a_ref[...] = g2[:, : NT // 2]
    g2b_ref[...] = g2[:, NT // 2:]


_dense2 = pl.pallas_call(
    _dense2_body,
    out_shape=[
        jax.ShapeDtypeStruct((N, NT // 2), jnp.float32),
        jax.ShapeDtypeStruct((N, NT // 2), jnp.float32),
    ],
)


def _final_body(p_ref, dinv_ref, b2_ref, o_ref):
    p = jnp.concatenate([p_ref[0], p_ref[1]], axis=1)
    o_ref[...] = p * dinv_ref[...] + b2_ref[...][None, :]


_final = pl.pallas_call(
    _final_body,
    out_shape=jax.ShapeDtypeStruct((N, NT), jnp.float32),
)


def kernel(x, edge_index, W1, b1, W2, b2):
    row = edge_index[0]
    col = edge_index[1]
    col_d = col.reshape(NW, NBLK, EB)
    row_s = row.reshape(NS, ABLK, EB)
    col_s = col.reshape(NS, ABLK, EB)
    hist = _deg_kernel(col_d)
    ga, gb, dinv = _dense1(x, W1, hist)
    p1 = _agg1(ga, gb, row_s, col_s)
    g2a, g2b = _dense2(p1, dinv, b1, W2)
    p2 = _agg2(g2a, g2b, row_s, col_s)
    return _final(p2, dinv, b2)


# speedup vs baseline: 30.6403x; 1.0814x over previous
import functools

import jax
import jax.numpy as jnp
from jax import lax
from jax.experimental import pallas as pl
from jax.experimental.pallas import tpu as pltpu
from jax.experimental.pallas import tpu_sc as plsc

N = 10000
E = 320000
NF = 128
NH = 128
NT = 64

NC, NS, L = 2, 16, 16
NW = NC * NS
EPW = E // NW
EB = 80
NBLK = EPW // EB
EPS = E // NS
ABLK = EPS // EB
S_ = 9
G_ = 4
RPT = 624
ZR = 48
TAIL0 = NS * RPT
TAILR = N - TAIL0

_MESH = dict(core_axis_name="c", subcore_axis_name="s")


@functools.partial(
    pl.kernel,
    out_type=jax.ShapeDtypeStruct((NW, N), jnp.float32),
    mesh=plsc.VectorSubcoreMesh(**_MESH),
    compiler_params=pltpu.CompilerParams(needs_layout_passes=False,
                                         use_tc_tiling_on_sc=False),
    scratch_types=[
        pltpu.VMEM((NBLK, EB), jnp.int32),
        pltpu.VMEM((N,), jnp.float32),
        pltpu.SemaphoreType.DMA,
    ],
)
def _deg_kernel(ei_hbm, hist_out, cidx_v, hist_v, sem):
    cid = lax.axis_index("c")
    sid = lax.axis_index("s")
    wid = cid * NS + sid
    idx_cp = pltpu.async_copy(ei_hbm.at[1, sid, pl.ds(cid * NBLK, NBLK)],
                              cidx_v, sem)
    z = jnp.zeros((L,), jnp.float32)

    def zero_blk(i, _):
        hist_v[pl.ds(i * L, L)] = z
        return 0

    lax.fori_loop(0, N // L, zero_blk, 0)
    idx_cp.wait()

    ones = jnp.full((L,), 1.0, jnp.float32)

    def blk(j, _):
        for g in range(EB // L):
            idx = cidx_v[j, pl.ds(g * L, L)]
            plsc.addupdate_scatter(hist_v, [idx], ones)
        return 0

    lax.fori_loop(0, NBLK, blk, 0)
    pltpu.sync_copy(hist_v, hist_out.at[wid])


def _make_agg(D):
    @functools.partial(
        pl.kernel,
        out_type=jax.ShapeDtypeStruct((NC, N, D), jnp.float32),
        mesh=plsc.VectorSubcoreMesh(**_MESH),
        compiler_params=pltpu.CompilerParams(use_tc_tiling_on_sc=False),
        scratch_types=[
            pltpu.VMEM_SHARED((N, D), jnp.float32),
            pltpu.VMEM((ABLK, EB), jnp.int32),
            pltpu.VMEM((ABLK, EB), jnp.int32),
            pltpu.VMEM((S_, EB, D), jnp.float32),
            pltpu.SemaphoreType.DMA((S_,)),
            pltpu.SemaphoreType.DMA((S_,)),
            pltpu.SemaphoreType.DMA,
            pltpu.SemaphoreType.DMA,
        ],
    )
    def agg(ga_hbm, gb_hbm, ei_hbm, out_hbm, acc_sh, ridx_v, cidx_v, rows_v,
            gsem, ssem, isem, jsem):
        cid = lax.axis_index("c")
        sid = lax.axis_index("s")
        rcp = pltpu.async_copy(ei_hbm.at[0, sid], ridx_v, isem)
        ccp = pltpu.async_copy(ei_hbm.at[1, sid], cidx_v, jsem)
        z = jnp.zeros((L,), jnp.float32)

        def zfill(r, _):
            for c0 in range(D // L):
                rows_v[0, r, pl.ds(c0 * L, L)] = z
            return 0

        lax.fori_loop(0, EB, zfill, 0)
        row0 = sid * RPT
        ZT = RPT - (RPT // EB) * EB

        def zero_blk(i, _):
            pltpu.sync_copy(rows_v.at[0], acc_sh.at[pl.ds(row0 + i * EB, EB)])
            return 0

        lax.fori_loop(0, RPT // EB, zero_blk, 0)
        pltpu.sync_copy(rows_v.at[0, pl.ds(0, ZT)],
                        acc_sh.at[pl.ds(row0 + (RPT // EB) * EB, ZT)])

        @pl.when(sid == 0)
        def _():
            pltpu.sync_copy(rows_v.at[0, pl.ds(0, TAILR)],
                            acc_sh.at[pl.ds(TAIL0, TAILR)])

        rcp.wait()
        ccp.wait()

        def gstart(j, s):
            @pl.when(cid == 0)
            def _():
                pltpu.async_copy(ga_hbm.at[ridx_v.at[j]], rows_v.at[s],
                                 gsem.at[s])

            @pl.when(cid == 1)
            def _():
                pltpu.async_copy(gb_hbm.at[ridx_v.at[j]], rows_v.at[s],
                                 gsem.at[s])

        def gwait(j, s):
            pltpu.make_async_copy(ga_hbm.at[ridx_v.at[j]], rows_v.at[s],
                                  gsem.at[s]).wait()

        def sstart(j, s):
            pltpu.async_copy(rows_v.at[s], acc_sh.at[cidx_v.at[j]],
                             ssem.at[s], add=True)

        def swait(j, s):
            pltpu.make_async_copy(rows_v.at[s], acc_sh.at[cidx_v.at[j]],
                                  ssem.at[s]).wait()

        for j in range(G_):
            gstart(j, j)
        plsc.subcore_barrier()

        def step(j, _):
            s = lax.rem(j, S_)
            gwait(j, s)
            sstart(j, s)

            @pl.when(j + G_ < ABLK)
            def _():
                s2 = lax.rem(j + G_, S_)

                @pl.when(j >= S_ - G_)
                def _():
                    swait(j - (S_ - G_), s2)

                gstart(j + G_, s2)

            return 0

        lax.fori_loop(0, ABLK, step, 0)

        def drain(k, _):
            j = ABLK - S_ + k
            swait(j, lax.rem(j, S_))
            return 0

        lax.fori_loop(0, S_, drain, 0)
        plsc.subcore_barrier()

        def copy_blk(i, _):
            r = row0 + i * EB
            pltpu.sync_copy(acc_sh.at[pl.ds(r, EB)],
                            out_hbm.at[cid, pl.ds(r, EB)])
            return 0

        lax.fori_loop(0, RPT // EB, copy_blk, 0)
        rz = row0 + (RPT // EB) * EB
        pltpu.sync_copy(acc_sh.at[pl.ds(rz, ZT)],
                        out_hbm.at[cid, pl.ds(rz, ZT)])

        @pl.when(sid == 0)
        def _():
            pltpu.sync_copy(acc_sh.at[pl.ds(TAIL0, TAILR)],
                            out_hbm.at[cid, pl.ds(TAIL0, TAILR)])

    return agg


_agg1 = _make_agg(NH // 2)
_agg2 = _make_agg(NT // 2)


def _dense1_body(x_ref, w_ref, hist_ref, ga_ref, gb_ref, dinv_ref):
    deg = jnp.sum(hist_ref[...], axis=0)
    dinv = jnp.where(deg > 0, 1.0 / jnp.sqrt(deg), 0.0)[:, None]
    g = jnp.dot(x_ref[...], w_ref[...],
                preferred_element_type=jnp.float32) * dinv
    ga_ref[...] = g[:, : NH // 2]
    gb_ref[...] = g[:, NH // 2:]
    dinv_ref[...] = dinv


_dense1 = pl.pallas_call(
    _dense1_body,
    out_shape=[
        jax.ShapeDtypeStruct((N, NH // 2), jnp.float32),
        jax.ShapeDtypeStruct((N, NH // 2), jnp.float32),
        jax.ShapeDtypeStruct((N, 1), jnp.float32),
    ],
)


def _dense2_body(p_ref, dinv_ref, b1_ref, w2_ref, g2a_ref, g2b_ref):
    dinv = dinv_ref[...]
    p = jnp.concatenate([p_ref[0], p_ref[1]], axis=1)
    h = jnp.maximum(p * dinv + b1_ref[...][None, :], 0.0)
    g2 = jnp.dot(h, w2_ref[...], preferred_element_type=jnp.float32) * dinv
    g2a_ref[...] = g2[:, : NT // 2]
    g2b_ref[...] = g2[:, NT // 2:]


_dense2 = pl.pallas_call(
    _dense2_body,
    out_shape=[
        jax.ShapeDtypeStruct((N, NT // 2), jnp.float32),
        jax.ShapeDtypeStruct((N, NT // 2), jnp.float32),
    ],
)


def _final_body(p_ref, dinv_ref, b2_ref, o_ref):
    p = jnp.concatenate([p_ref[0], p_ref[1]], axis=1)
    o_ref[...] = p * dinv_ref[...] + b2_ref[...][None, :]


_final = pl.pallas_call(
    _final_body,
    out_shape=jax.ShapeDtypeStruct((N, NT), jnp.float32),
)


def kernel(x, edge_index, W1, b1, W2, b2):
    ei4 = edge_index.reshape(2, NS, ABLK, EB)
    hist = _deg_kernel(ei4)
    ga, gb, dinv = _dense1(x, W1, hist)
    p1 = _agg1(ga, gb, ei4)
    g2a, g2b = _dense2(p1, dinv, b1, W2)
    p2 = _agg2(g2a, g2b, ei4)
    return _final(p2, dinv, b2)


# speedup vs baseline: 33.1649x; 1.0824x over previous
import functools

import jax
import jax.numpy as jnp
from jax import lax
from jax.experimental import pallas as pl
from jax.experimental.pallas import tpu as pltpu
from jax.experimental.pallas import tpu_sc as plsc

N = 10000
E = 320000
NF = 128
NH = 128
NT = 64

NC, NS, L = 2, 16, 16
NW = NC * NS
EPW = E // NW
EB = 80
NBLK = EPW // EB
EPS = E // NS
ABLK = EPS // EB
S_ = 9
G_ = 4
RPT = 624
ZR = 48
TAIL0 = NS * RPT
TAILR = N - TAIL0

_MESH = dict(core_axis_name="c", subcore_axis_name="s")


@functools.partial(
    pl.kernel,
    out_type=jax.ShapeDtypeStruct((NW, N), jnp.float32),
    mesh=plsc.VectorSubcoreMesh(**_MESH),
    compiler_params=pltpu.CompilerParams(needs_layout_passes=False,
                                         use_tc_tiling_on_sc=False),
    scratch_types=[
        pltpu.VMEM((NBLK, EB), jnp.int32),
        pltpu.VMEM((N,), jnp.float32),
        pltpu.SemaphoreType.DMA,
    ],
)
def _deg_kernel(ei_hbm, hist_out, cidx_v, hist_v, sem):
    cid = lax.axis_index("c")
    sid = lax.axis_index("s")
    wid = cid * NS + sid
    idx_cp = pltpu.async_copy(ei_hbm.at[1, sid, pl.ds(cid * NBLK, NBLK)],
                              cidx_v, sem)
    z = jnp.zeros((L,), jnp.float32)

    def zero_blk(i, _):
        hist_v[pl.ds(i * L, L)] = z
        return 0

    lax.fori_loop(0, N // L, zero_blk, 0)
    idx_cp.wait()

    ones = jnp.full((L,), 1.0, jnp.float32)

    def blk(j, _):
        for g in range(EB // L):
            idx = cidx_v[j, pl.ds(g * L, L)]
            plsc.addupdate_scatter(hist_v, [idx], ones)
        return 0

    lax.fori_loop(0, NBLK, blk, 0)
    pltpu.sync_copy(hist_v, hist_out.at[wid])


D1 = NH // 2


@functools.partial(
    pl.kernel,
    out_type=jax.ShapeDtypeStruct((N, NH), jnp.float32),
    mesh=plsc.VectorSubcoreMesh(**_MESH),
    compiler_params=pltpu.CompilerParams(use_tc_tiling_on_sc=False),
    scratch_types=[
        pltpu.VMEM_SHARED((N, D1), jnp.float32),
        pltpu.VMEM((ABLK, EB), jnp.int32),
        pltpu.VMEM((ABLK, EB), jnp.int32),
        pltpu.VMEM((S_, EB, D1), jnp.float32),
        pltpu.SemaphoreType.DMA((S_,)),
        pltpu.SemaphoreType.DMA((S_,)),
        pltpu.SemaphoreType.DMA,
        pltpu.SemaphoreType.DMA,
    ],
)
def _agg1(gv_hbm, ei_hbm, out_hbm, acc_sh, ridx_v, cidx_v, rows_v,
          gsem, ssem, isem, jsem):
    cid = lax.axis_index("c")
    sid = lax.axis_index("s")
    rcp = pltpu.async_copy(ei_hbm.at[0, sid], ridx_v, isem)
    ccp = pltpu.async_copy(ei_hbm.at[1, sid], cidx_v, jsem)
    z = jnp.zeros((L,), jnp.float32)

    def zfill(r, _):
        for c0 in range(D1 // L):
            rows_v[0, r, pl.ds(c0 * L, L)] = z
        return 0

    lax.fori_loop(0, EB, zfill, 0)
    row0 = sid * RPT
    ZT = RPT - (RPT // EB) * EB

    def zero_blk(i, _):
        pltpu.sync_copy(rows_v.at[0], acc_sh.at[pl.ds(row0 + i * EB, EB)])
        return 0

    lax.fori_loop(0, RPT // EB, zero_blk, 0)
    pltpu.sync_copy(rows_v.at[0, pl.ds(0, ZT)],
                    acc_sh.at[pl.ds(row0 + (RPT // EB) * EB, ZT)])

    @pl.when(sid == 0)
    def _():
        pltpu.sync_copy(rows_v.at[0, pl.ds(0, TAILR)],
                        acc_sh.at[pl.ds(TAIL0, TAILR)])

    rcp.wait()
    cidv = jnp.full((L,), cid, jnp.int32)

    def xform(j, _):
        for gg in range(EB // L):
            v = ridx_v[j, pl.ds(gg * L, L)]
            ridx_v[j, pl.ds(gg * L, L)] = v * 2 + cidv
        return 0

    lax.fori_loop(0, ABLK, xform, 0)
    ccp.wait()

    def gstart(j, s):
        pltpu.async_copy(gv_hbm.at[ridx_v.at[j]], rows_v.at[s], gsem.at[s])

    def gwait(j, s):
        pltpu.make_async_copy(gv_hbm.at[ridx_v.at[j]], rows_v.at[s],
                              gsem.at[s]).wait()

    def sstart(j, s):
        pltpu.async_copy(rows_v.at[s], acc_sh.at[cidx_v.at[j]],
                         ssem.at[s], add=True)

    def swait(j, s):
        pltpu.make_async_copy(rows_v.at[s], acc_sh.at[cidx_v.at[j]],
                              ssem.at[s]).wait()

    for j in range(G_):
        gstart(j, j)
    plsc.subcore_barrier()

    def step(j, _):
        s = lax.rem(j, S_)
        gwait(j, s)
        sstart(j, s)

        @pl.when(j + G_ < ABLK)
        def _():
            s2 = lax.rem(j + G_, S_)

            @pl.when(j >= S_ - G_)
            def _():
                swait(j - (S_ - G_), s2)

            gstart(j + G_, s2)

        return 0

    lax.fori_loop(0, ABLK, step, 0)

    def drain(k, _):
        j = ABLK - S_ + k
        swait(j, lax.rem(j, S_))
        return 0

    lax.fori_loop(0, S_, drain, 0)
    plsc.subcore_barrier()

    def copy_blk(i, _):
        r = row0 + i * EB
        pltpu.sync_copy(acc_sh.at[pl.ds(r, EB)],
                        out_hbm.at[pl.ds(r, EB), pl.ds(cid * D1, D1)])
        return 0

    lax.fori_loop(0, RPT // EB, copy_blk, 0)
    rz = row0 + (RPT // EB) * EB
    pltpu.sync_copy(acc_sh.at[pl.ds(rz, ZT)],
                    out_hbm.at[pl.ds(rz, ZT), pl.ds(cid * D1, D1)])

    @pl.when(sid == 0)
    def _():
        pltpu.sync_copy(acc_sh.at[pl.ds(TAIL0, TAILR)],
                        out_hbm.at[pl.ds(TAIL0, TAILR), pl.ds(cid * D1, D1)])


def _make_agg(D):
    @functools.partial(
        pl.kernel,
        out_type=jax.ShapeDtypeStruct((NC, N, D), jnp.float32),
        mesh=plsc.VectorSubcoreMesh(**_MESH),
        compiler_params=pltpu.CompilerParams(use_tc_tiling_on_sc=False),
        scratch_types=[
            pltpu.VMEM_SHARED((N, D), jnp.float32),
            pltpu.VMEM((ABLK, EB), jnp.int32),
            pltpu.VMEM((ABLK, EB), jnp.int32),
            pltpu.VMEM((S_, EB, D), jnp.float32),
            pltpu.SemaphoreType.DMA((S_,)),
            pltpu.SemaphoreType.DMA((S_,)),
            pltpu.SemaphoreType.DMA,
            pltpu.SemaphoreType.DMA,
        ],
    )
    def agg(ga_hbm, gb_hbm, ei_hbm, out_hbm, acc_sh, ridx_v, cidx_v, rows_v,
            gsem, ssem, isem, jsem):
        cid = lax.axis_index("c")
        sid = lax.axis_index("s")
        rcp = pltpu.async_copy(ei_hbm.at[0, sid], ridx_v, isem)
        ccp = pltpu.async_copy(ei_hbm.at[1, sid], cidx_v, jsem)
        z = jnp.zeros((L,), jnp.float32)

        def zfill(r, _):
            for c0 in range(D // L):
                rows_v[0, r, pl.ds(c0 * L, L)] = z
            return 0

        lax.fori_loop(0, EB, zfill, 0)
        row0 = sid * RPT
        ZT = RPT - (RPT // EB) * EB

        def zero_blk(i, _):
            pltpu.sync_copy(rows_v.at[0], acc_sh.at[pl.ds(row0 + i * EB, EB)])
            return 0

        lax.fori_loop(0, RPT // EB, zero_blk, 0)
        pltpu.sync_copy(rows_v.at[0, pl.ds(0, ZT)],
                        acc_sh.at[pl.ds(row0 + (RPT // EB) * EB, ZT)])

        @pl.when(sid == 0)
        def _():
            pltpu.sync_copy(rows_v.at[0, pl.ds(0, TAILR)],
                            acc_sh.at[pl.ds(TAIL0, TAILR)])

        rcp.wait()
        ccp.wait()

        def gstart(j, s):
            @pl.when(cid == 0)
            def _():
                pltpu.async_copy(ga_hbm.at[ridx_v.at[j]], rows_v.at[s],
                                 gsem.at[s])

            @pl.when(cid == 1)
            def _():
                pltpu.async_copy(gb_hbm.at[ridx_v.at[j]], rows_v.at[s],
                                 gsem.at[s])

        def gwait(j, s):
            pltpu.make_async_copy(ga_hbm.at[ridx_v.at[j]], rows_v.at[s],
                                  gsem.at[s]).wait()

        def sstart(j, s):
            pltpu.async_copy(rows_v.at[s], acc_sh.at[cidx_v.at[j]],
                             ssem.at[s], add=True)

        def swait(j, s):
            pltpu.make_async_copy(rows_v.at[s], acc_sh.at[cidx_v.at[j]],
                                  ssem.at[s]).wait()

        for j in range(G_):
            gstart(j, j)
        plsc.subcore_barrier()

        def step(j, _):
            s = lax.rem(j, S_)
            gwait(j, s)
            sstart(j, s)

            @pl.when(j + G_ < ABLK)
            def _():
                s2 = lax.rem(j + G_, S_)

                @pl.when(j >= S_ - G_)
                def _():
                    swait(j - (S_ - G_), s2)

                gstart(j + G_, s2)

            return 0

        lax.fori_loop(0, ABLK, step, 0)

        def drain(k, _):
            j = ABLK - S_ + k
            swait(j, lax.rem(j, S_))
            return 0

        lax.fori_loop(0, S_, drain, 0)
        plsc.subcore_barrier()

        def copy_blk(i, _):
            r = row0 + i * EB
            pltpu.sync_copy(acc_sh.at[pl.ds(r, EB)],
                            out_hbm.at[cid, pl.ds(r, EB)])
            return 0

        lax.fori_loop(0, RPT // EB, copy_blk, 0)
        rz = row0 + (RPT // EB) * EB
        pltpu.sync_copy(acc_sh.at[pl.ds(rz, ZT)],
                        out_hbm.at[cid, pl.ds(rz, ZT)])

        @pl.when(sid == 0)
        def _():
            pltpu.sync_copy(acc_sh.at[pl.ds(TAIL0, TAILR)],
                            out_hbm.at[cid, pl.ds(TAIL0, TAILR)])

    return agg


_agg2 = _make_agg(NT // 2)


def _dense1_body(x_ref, w_ref, hist_ref, g_ref, dinv_ref):
    deg = jnp.sum(hist_ref[...], axis=0)
    dinv = jnp.where(deg > 0, 1.0 / jnp.sqrt(deg), 0.0)[:, None]
    g_ref[...] = jnp.dot(x_ref[...], w_ref[...],
                         preferred_element_type=jnp.float32) * dinv
    dinv_ref[...] = dinv


_dense1 = pl.pallas_call(
    _dense1_body,
    out_shape=[
        jax.ShapeDtypeStruct((N, NH), jnp.float32),
        jax.ShapeDtypeStruct((N, 1), jnp.float32),
    ],
)


def _dense2_body(p_ref, dinv_ref, b1_ref, w2_ref, g2a_ref, g2b_ref):
    dinv = dinv_ref[...]
    p = p_ref[...]
    h = jnp.maximum(p * dinv + b1_ref[...][None, :], 0.0)
    g2 = jnp.dot(h, w2_ref[...], preferred_element_type=jnp.float32) * dinv
    g2a_ref[...] = g2[:, : NT // 2]
    g2b_ref[...] = g2[:, NT // 2:]


_dense2 = pl.pallas_call(
    _dense2_body,
    out_shape=[
        jax.ShapeDtypeStruct((N, NT // 2), jnp.float32),
        jax.ShapeDtypeStruct((N, NT // 2), jnp.float32),
    ],
)


def _final_body(p_ref, dinv_ref, b2_ref, o_ref):
    p = jnp.concatenate([p_ref[0], p_ref[1]], axis=1)
    o_ref[...] = p * dinv_ref[...] + b2_ref[...][None, :]


_final = pl.pallas_call(
    _final_body,
    out_shape=jax.ShapeDtypeStruct((N, NT), jnp.float32),
)


def kernel(x, edge_index, W1, b1, W2, b2):
    ei4 = edge_index.reshape(2, NS, ABLK, EB)
    hist = _deg_kernel(ei4)
    g1, dinv = _dense1(x, W1, hist)
    p1 = _agg1(g1.reshape(2 * N, NH // 2), ei4)
    g2a, g2b = _dense2(p1, dinv, b1, W2)
    p2 = _agg2(g2a, g2b, ei4)
    return _final(p2, dinv, b2)


# speedup vs baseline: 36.6708x; 1.1057x over previous
import functools

import jax
import jax.numpy as jnp
from jax import lax
from jax.experimental import pallas as pl
from jax.experimental.pallas import tpu as pltpu
from jax.experimental.pallas import tpu_sc as plsc

N = 10000
E = 320000
NF = 128
NH = 128
NT = 64

NC, NS, L = 2, 16, 16
NW = NC * NS
EPW = E // NW
EB = 80
NBLK = EPW // EB
EPS = E // NS
ABLK = EPS // EB
S_ = 9
G_ = 4
RPT = 624
ZR = 48
TAIL0 = NS * RPT
TAILR = N - TAIL0

_MESH = dict(core_axis_name="c", subcore_axis_name="s")


@functools.partial(
    pl.kernel,
    out_type=jax.ShapeDtypeStruct((NW, N), jnp.float32),
    mesh=plsc.VectorSubcoreMesh(**_MESH),
    compiler_params=pltpu.CompilerParams(needs_layout_passes=False,
                                         use_tc_tiling_on_sc=False),
    scratch_types=[
        pltpu.VMEM((NBLK, EB), jnp.int32),
        pltpu.VMEM((N,), jnp.float32),
        pltpu.SemaphoreType.DMA,
    ],
)
def _deg_kernel(ei_hbm, hist_out, cidx_v, hist_v, sem):
    cid = lax.axis_index("c")
    sid = lax.axis_index("s")
    wid = cid * NS + sid
    idx_cp = pltpu.async_copy(ei_hbm.at[1, sid, pl.ds(cid * NBLK, NBLK)],
                              cidx_v, sem)
    z = jnp.zeros((L,), jnp.float32)

    def zero_blk(i, _):
        hist_v[pl.ds(i * L, L)] = z
        return 0

    lax.fori_loop(0, N // L, zero_blk, 0)
    idx_cp.wait()

    ones = jnp.full((L,), 1.0, jnp.float32)

    def blk(j, _):
        for g in range(EB // L):
            idx = cidx_v[j, pl.ds(g * L, L)]
            plsc.addupdate_scatter(hist_v, [idx], ones)
        return 0

    lax.fori_loop(0, NBLK, blk, 0)
    pltpu.sync_copy(hist_v, hist_out.at[wid])


D1 = NH // 2


@functools.partial(
    pl.kernel,
    out_type=jax.ShapeDtypeStruct((N, NH), jnp.float32),
    mesh=plsc.VectorSubcoreMesh(**_MESH),
    compiler_params=pltpu.CompilerParams(use_tc_tiling_on_sc=False),
    scratch_types=[
        pltpu.VMEM_SHARED((N, D1), jnp.float32),
        pltpu.VMEM((ABLK, EB), jnp.int32),
        pltpu.VMEM((ABLK, EB), jnp.int32),
        pltpu.VMEM((S_, EB, D1), jnp.float32),
        pltpu.SemaphoreType.DMA((S_,)),
        pltpu.SemaphoreType.DMA((S_,)),
        pltpu.SemaphoreType.DMA,
        pltpu.SemaphoreType.DMA,
    ],
)
def _agg1(gv_hbm, ei_hbm, out_hbm, acc_sh, ridx_v, cidx_v, rows_v,
          gsem, ssem, isem, jsem):
    cid = lax.axis_index("c")
    sid = lax.axis_index("s")
    rcp = pltpu.async_copy(ei_hbm.at[0, sid], ridx_v, isem)
    ccp = pltpu.async_copy(ei_hbm.at[1, sid], cidx_v, jsem)
    z = jnp.zeros((L,), jnp.float32)

    def zfill(r, _):
        for c0 in range(D1 // L):
            rows_v[0, r, pl.ds(c0 * L, L)] = z
        return 0

    lax.fori_loop(0, EB, zfill, 0)
    row0 = sid * RPT
    ZT = RPT - (RPT // EB) * EB

    def zero_blk(i, _):
        pltpu.sync_copy(rows_v.at[0], acc_sh.at[pl.ds(row0 + i * EB, EB)])
        return 0

    lax.fori_loop(0, RPT // EB, zero_blk, 0)
    pltpu.sync_copy(rows_v.at[0, pl.ds(0, ZT)],
                    acc_sh.at[pl.ds(row0 + (RPT // EB) * EB, ZT)])

    @pl.when(sid == 0)
    def _():
        pltpu.sync_copy(rows_v.at[0, pl.ds(0, TAILR)],
                        acc_sh.at[pl.ds(TAIL0, TAILR)])

    rcp.wait()
    cidv = jnp.full((L,), cid, jnp.int32)

    def xform(j, _):
        for gg in range(EB // L):
            v = ridx_v[j, pl.ds(gg * L, L)]
            ridx_v[j, pl.ds(gg * L, L)] = v * 2 + cidv
        return 0

    lax.fori_loop(0, ABLK, xform, 0)
    ccp.wait()

    def gstart(j, s):
        pltpu.async_copy(gv_hbm.at[ridx_v.at[j]], rows_v.at[s], gsem.at[s])

    def gwait(j, s):
        pltpu.make_async_copy(gv_hbm.at[ridx_v.at[j]], rows_v.at[s],
                              gsem.at[s]).wait()

    def sstart(j, s):
        pltpu.async_copy(rows_v.at[s], acc_sh.at[cidx_v.at[j]],
                         ssem.at[s], add=True)

    def swait(j, s):
        pltpu.make_async_copy(rows_v.at[s], acc_sh.at[cidx_v.at[j]],
                              ssem.at[s]).wait()

    for j in range(G_):
        gstart(j, j)
    plsc.subcore_barrier()

    def step(j, _):
        s = lax.rem(j, S_)
        gwait(j, s)
        sstart(j, s)

        @pl.when(j + G_ < ABLK)
        def _():
            s2 = lax.rem(j + G_, S_)

            @pl.when(j >= S_ - G_)
            def _():
                swait(j - (S_ - G_), s2)

            gstart(j + G_, s2)

        return 0

    lax.fori_loop(0, ABLK, step, 0)

    def drain(k, _):
        j = ABLK - S_ + k
        swait(j, lax.rem(j, S_))
        return 0

    lax.fori_loop(0, S_, drain, 0)
    plsc.subcore_barrier()

    def copy_blk(i, _):
        r = row0 + i * EB
        pltpu.sync_copy(acc_sh.at[pl.ds(r, EB)],
                        out_hbm.at[pl.ds(r, EB), pl.ds(cid * D1, D1)])
        return 0

    lax.fori_loop(0, RPT // EB, copy_blk, 0)
    rz = row0 + (RPT // EB) * EB
    pltpu.sync_copy(acc_sh.at[pl.ds(rz, ZT)],
                    out_hbm.at[pl.ds(rz, ZT), pl.ds(cid * D1, D1)])

    @pl.when(sid == 0)
    def _():
        pltpu.sync_copy(acc_sh.at[pl.ds(TAIL0, TAILR)],
                        out_hbm.at[pl.ds(TAIL0, TAILR), pl.ds(cid * D1, D1)])


D2 = NT
ABLK2 = EPW // EB


@functools.partial(
    pl.kernel,
    out_type=jax.ShapeDtypeStruct((N, 2 * D2), jnp.float32),
    mesh=plsc.VectorSubcoreMesh(**_MESH),
    compiler_params=pltpu.CompilerParams(use_tc_tiling_on_sc=False),
    scratch_types=[
        pltpu.VMEM_SHARED((N, D2), jnp.float32),
        pltpu.VMEM((ABLK, EB), jnp.int32),
        pltpu.VMEM((ABLK, EB), jnp.int32),
        pltpu.VMEM((S_, EB, D2), jnp.float32),
        pltpu.SemaphoreType.DMA((S_,)),
        pltpu.SemaphoreType.DMA((S_,)),
        pltpu.SemaphoreType.DMA,
        pltpu.SemaphoreType.DMA,
    ],
)
def _agg2(gv_hbm, ei_hbm, out_hbm, acc_sh, ridx_v, cidx_v, rows_v,
          gsem, ssem, isem, jsem):
    cid = lax.axis_index("c")
    sid = lax.axis_index("s")
    rcp = pltpu.async_copy(ei_hbm.at[0, sid], ridx_v, isem)
    ccp = pltpu.async_copy(ei_hbm.at[1, sid], cidx_v, jsem)
    w0 = cid * ABLK2
    z = jnp.zeros((L,), jnp.float32)

    def zfill(r, _):
        for c0 in range(D2 // L):
            rows_v[0, r, pl.ds(c0 * L, L)] = z
        return 0

    lax.fori_loop(0, EB, zfill, 0)
    row0 = sid * RPT
    ZT = RPT - (RPT // EB) * EB

    def zero_blk(i, _):
        pltpu.sync_copy(rows_v.at[0], acc_sh.at[pl.ds(row0 + i * EB, EB)])
        return 0

    lax.fori_loop(0, RPT // EB, zero_blk, 0)
    pltpu.sync_copy(rows_v.at[0, pl.ds(0, ZT)],
                    acc_sh.at[pl.ds(row0 + (RPT // EB) * EB, ZT)])

    @pl.when(sid == 0)
    def _():
        pltpu.sync_copy(rows_v.at[0, pl.ds(0, TAILR)],
                        acc_sh.at[pl.ds(TAIL0, TAILR)])

    rcp.wait()
    two = jnp.full((L,), 2, jnp.int32)

    def xform(j, _):
        for gg in range(EB // L):
            v = ridx_v[w0 + j, pl.ds(gg * L, L)]
            ridx_v[w0 + j, pl.ds(gg * L, L)] = v * two
        return 0

    lax.fori_loop(0, ABLK2, xform, 0)
    ccp.wait()

    def gstart(j, s):
        pltpu.async_copy(gv_hbm.at[ridx_v.at[w0 + j]], rows_v.at[s],
                         gsem.at[s])

    def gwait(j, s):
        pltpu.make_async_copy(gv_hbm.at[ridx_v.at[w0 + j]], rows_v.at[s],
                              gsem.at[s]).wait()

    def sstart(j, s):
        pltpu.async_copy(rows_v.at[s], acc_sh.at[cidx_v.at[w0 + j]],
                         ssem.at[s], add=True)

    def swait(j, s):
        pltpu.make_async_copy(rows_v.at[s], acc_sh.at[cidx_v.at[w0 + j]],
                              ssem.at[s]).wait()

    for j in range(G_):
        gstart(j, j)
    plsc.subcore_barrier()

    def step(j, _):
        s = lax.rem(j, S_)
        gwait(j, s)
        sstart(j, s)

        @pl.when(j + G_ < ABLK2)
        def _():
            s2 = lax.rem(j + G_, S_)

            @pl.when(j >= S_ - G_)
            def _():
                swait(j - (S_ - G_), s2)

            gstart(j + G_, s2)

        return 0

    lax.fori_loop(0, ABLK2, step, 0)

    def drain(k, _):
        j = ABLK2 - S_ + k
        swait(j, lax.rem(j, S_))
        return 0

    lax.fori_loop(0, S_, drain, 0)
    plsc.subcore_barrier()

    def copy_blk(i, _):
        r = row0 + i * EB
        pltpu.sync_copy(acc_sh.at[pl.ds(r, EB)],
                        out_hbm.at[pl.ds(r, EB), pl.ds(cid * D2, D2)])
        return 0

    lax.fori_loop(0, RPT // EB, copy_blk, 0)
    rz = row0 + (RPT // EB) * EB
    pltpu.sync_copy(acc_sh.at[pl.ds(rz, ZT)],
                    out_hbm.at[pl.ds(rz, ZT), pl.ds(cid * D2, D2)])

    @pl.when(sid == 0)
    def _():
        pltpu.sync_copy(acc_sh.at[pl.ds(TAIL0, TAILR)],
                        out_hbm.at[pl.ds(TAIL0, TAILR), pl.ds(cid * D2, D2)])


def _dense1_body(x_ref, w_ref, hist_ref, g_ref, dinv_ref):
    deg = jnp.sum(hist_ref[...], axis=0)
    dinv = jnp.where(deg > 0, 1.0 / jnp.sqrt(deg), 0.0)[:, None]
    g_ref[...] = jnp.dot(x_ref[...], w_ref[...],
                         preferred_element_type=jnp.float32) * dinv
    dinv_ref[...] = dinv


_dense1 = pl.pallas_call(
    _dense1_body,
    out_shape=[
        jax.ShapeDtypeStruct((N, NH), jnp.float32),
        jax.ShapeDtypeStruct((N, 1), jnp.float32),
    ],
)


def _dense2_body(p_ref, dinv_ref, b1_ref, w2_ref, g2_ref):
    dinv = dinv_ref[...]
    p = p_ref[...]
    h = jnp.maximum(p * dinv + b1_ref[...][None, :], 0.0)
    g2 = jnp.dot(h, w2_ref[...], preferred_element_type=jnp.float32) * dinv
    g2_ref[...] = jnp.concatenate([g2, g2], axis=1)


_dense2 = pl.pallas_call(
    _dense2_body,
    out_shape=jax.ShapeDtypeStruct((N, 2 * NT), jnp.float32),
)


def _final_body(p_ref, dinv_ref, b2_ref, o_ref):
    p = p_ref[...]
    o_ref[...] = ((p[:, :NT] + p[:, NT:]) * dinv_ref[...]
                  + b2_ref[...][None, :])


_final = pl.pallas_call(
    _final_body,
    out_shape=jax.ShapeDtypeStruct((N, NT), jnp.float32),
)


def kernel(x, edge_index, W1, b1, W2, b2):
    ei4 = edge_index.reshape(2, NS, ABLK, EB)
    hist = _deg_kernel(ei4)
    g1, dinv = _dense1(x, W1, hist)
    p1 = _agg1(g1.reshape(2 * N, NH // 2), ei4)
    g2d = _dense2(p1, dinv, b1, W2)
    p2 = _agg2(g2d.reshape(2 * N, NT), ei4)
    return _final(p2, dinv, b2)
